# baseline (speedup 1.0000x reference)
import weakref

import jax
import jax.numpy as jnp
from jax import lax
from jax.experimental import pallas as pl
from jax.experimental.pallas import tpu as pltpu
from jax.experimental.pallas import tpu_sc as plsc

_NUM_EMB = 1_000_000
_DIM = 32
_PACK = 128 // _DIM
_BATCH = 16384
_NUM_CORES = 2
_NUM_SUBCORES = 16
_NUM_WORKERS = _NUM_CORES * _NUM_SUBCORES
_B_PER_W = _BATCH // _NUM_WORKERS

_mesh = plsc.VectorSubcoreMesh(core_axis_name="c", subcore_axis_name="s")


def _gather_body(table_hbm, idx_hbm, out_hbm, idx_v, line_v, rows_v, sem):
    wid = lax.axis_index("s") * _NUM_CORES + lax.axis_index("c")
    base = wid * _B_PER_W

    pltpu.sync_copy(idx_hbm.at[pl.ds(base, _B_PER_W)], idx_v)
    for k in range(_B_PER_W // 16):
        line_v[pl.ds(k * 16, 16)] = idx_v[pl.ds(k * 16, 16)] >> 2
    pltpu.async_copy(table_hbm.at[line_v], rows_v, sem).wait()

    def group_step(g, _):
        v = idx_v[pl.ds(g * 16, 16)]
        off = (v & 3) * _DIM
        for k in range(16):
            i = g * 16 + k
            lo = rows_v[i, pl.ds(off[k], 16)]
            hi = rows_v[i, pl.ds(off[k] + 16, 16)]
            rows_v[i, pl.ds(0, 16)] = lo
            rows_v[i, pl.ds(16, 16)] = hi
        return ()

    lax.fori_loop(0, _B_PER_W // 16, group_step, (), unroll=False)

    pltpu.sync_copy(rows_v, out_hbm.at[pl.ds(base, _B_PER_W)])


def _make_gather():
    return pl.kernel(
        _gather_body,
        mesh=_mesh,
        out_type=jax.ShapeDtypeStruct((_BATCH, _PACK * _DIM), jnp.float32),
        scratch_types=[
            pltpu.VMEM((_B_PER_W,), jnp.int32),
            pltpu.VMEM((_B_PER_W,), jnp.int32),
            pltpu.VMEM((_B_PER_W, _PACK * _DIM), jnp.float32),
            pltpu.SemaphoreType.DMA,
        ],
        compiler_params=pltpu.CompilerParams(use_tc_tiling_on_sc=True),
    )


@jax.jit
def _pack_table(embeds):
    return jnp.reshape(embeds, (_NUM_EMB // _PACK, _PACK * _DIM))


@jax.jit
def _gather(packed, input_index):
    out_lines = _make_gather()(packed, input_index.astype(jnp.int32))
    return out_lines[:, :_DIM]


_packed_cache: dict[int, tuple] = {}


def _packed_for(embeds):
    key = id(embeds)
    hit = _packed_cache.get(key)
    if hit is not None and hit[0]() is embeds:
        return hit[1]
    packed = _pack_table(embeds)
    if len(_packed_cache) >= 8:
        _packed_cache.clear()
    try:
        ref = weakref.ref(embeds)
    except TypeError:
        ref = (lambda e: (lambda: e))(embeds)
    _packed_cache[key] = (ref, packed)
    return packed


def kernel(input_index, embeds):
    return _gather(_packed_for(embeds), input_index)


# speedup vs baseline: 3.0428x; 3.0428x over previous
import jax
import jax.numpy as jnp
from jax import lax
from jax.experimental import pallas as pl
from jax.experimental.pallas import tpu as pltpu
from jax.experimental.pallas import tpu_sc as plsc

_NUM_EMB = 1_000_000
_DIM = 32
_BATCH = 16384
_NUM_CORES = 2
_NUM_SUBCORES = 16
_NUM_WORKERS = _NUM_CORES * _NUM_SUBCORES
_B_PER_W = _BATCH // _NUM_WORKERS
_GRP = 8
_SLIVER = 128

_mesh = plsc.VectorSubcoreMesh(core_axis_name="c", subcore_axis_name="s")


def _gather_body(table_hbm, idx_hbm, out_hbm, idx_v, fetch_v, stage_v, sem):
    wid = lax.axis_index("s") * _NUM_CORES + lax.axis_index("c")
    base = wid * _B_PER_W

    pltpu.sync_copy(idx_hbm.at[pl.ds(base, _B_PER_W)], idx_v)

    rows_lo = lax.iota(jnp.int32, 16)
    rows_hi = rows_lo + 16

    def group_step(g, _):
        v = idx_v[pl.ds(g * _GRP, 16)]
        j0 = (v >> 7) << 7
        lane = v & 127
        descs = []
        for k in range(_GRP):
            descs.append(
                pltpu.async_copy(
                    table_hbm.at[
                        pl.ds(0, _DIM),
                        pl.ds(pl.multiple_of(j0[k], 128), _SLIVER),
                    ],
                    fetch_v.at[k],
                    sem,
                )
            )
        for d in descs:
            d.wait()
        for k in range(_GRP):
            sel_k = jnp.full((16,), k, jnp.int32)
            cols = jnp.full((16,), lane[k], jnp.int32)
            lo = plsc.load_gather(fetch_v, [sel_k, rows_lo, cols])
            hi = plsc.load_gather(fetch_v, [sel_k, rows_hi, cols])
            i = g * _GRP + k
            stage_v[i, pl.ds(0, 16)] = lo
            stage_v[i, pl.ds(16, 16)] = hi
        return ()

    lax.fori_loop(0, _B_PER_W // _GRP, group_step, (), unroll=False)

    pltpu.sync_copy(stage_v, out_hbm.at[pl.ds(base, _B_PER_W)])


@jax.jit
def kernel(input_index, embeds):
    gather = pl.kernel(
        _gather_body,
        mesh=_mesh,
        out_type=jax.ShapeDtypeStruct((_BATCH, _DIM), jnp.float32),
        scratch_types=[
            pltpu.VMEM((_B_PER_W,), jnp.int32),
            pltpu.VMEM((_GRP, _DIM, _SLIVER), jnp.float32),
            pltpu.VMEM((_B_PER_W, _DIM), jnp.float32),
            pltpu.SemaphoreType.DMA,
        ],
        compiler_params=pltpu.CompilerParams(
            use_tc_tiling_on_sc=True, needs_layout_passes=False
        ),
    )
    return gather(embeds.T, input_index.astype(jnp.int32))


# speedup vs baseline: 3.6841x; 1.2108x over previous
import jax
import jax.numpy as jnp
from jax import lax
from jax.experimental import pallas as pl
from jax.experimental.pallas import tpu as pltpu
from jax.experimental.pallas import tpu_sc as plsc

_NUM_EMB = 1_000_000
_DIM = 32
_BATCH = 16384
_NUM_CORES = 2
_NUM_SUBCORES = 16
_NUM_WORKERS = _NUM_CORES * _NUM_SUBCORES
_B_PER_W = _BATCH // _NUM_WORKERS
_GRP = 8
_NGRP = _B_PER_W // _GRP
_SLIVER = 128

_mesh = plsc.VectorSubcoreMesh(core_axis_name="c", subcore_axis_name="s")


def _gather_body(table_hbm, idx_hbm, out_hbm, idx_v, fetch_v, stage_v,
                 sem_a, sem_b):
    wid = lax.axis_index("s") * _NUM_CORES + lax.axis_index("c")
    base = wid * _B_PER_W
    sems = (sem_a, sem_b)

    pltpu.sync_copy(idx_hbm.at[pl.ds(base, _B_PER_W)], idx_v)

    rows_lo = lax.iota(jnp.int32, 16)
    rows_hi = rows_lo + 16

    def issue_fetch(g, buf):
        v = idx_v[pl.ds(g * _GRP, 16)]
        j0 = (v >> 7) << 7
        for k in range(_GRP):
            pltpu.async_copy(
                table_hbm.at[
                    pl.ds(0, _DIM),
                    pl.ds(pl.multiple_of(j0[k], 128), _SLIVER),
                ],
                fetch_v.at[buf, k],
                sems[buf],
            )

    def drain_fetch(buf):
        for k in range(_GRP):
            pltpu.make_async_copy(
                table_hbm.at[pl.ds(0, _DIM), pl.ds(0, _SLIVER)],
                fetch_v.at[buf, k],
                sems[buf],
            ).wait()

    def extract(g, buf):
        v = idx_v[pl.ds(g * _GRP, 16)]
        lane = v & 127
        for k in range(_GRP):
            sel_k = jnp.full((16,), k, jnp.int32)
            cols = jnp.full((16,), lane[k], jnp.int32)
            lo = plsc.load_gather(fetch_v.at[buf], [sel_k, rows_lo, cols])
            hi = plsc.load_gather(fetch_v.at[buf], [sel_k, rows_hi, cols])
            stage_v[buf, k, pl.ds(0, 16)] = lo
            stage_v[buf, k, pl.ds(16, 16)] = hi
        pltpu.sync_copy(
            stage_v.at[buf], out_hbm.at[pl.ds(base + g * _GRP, _GRP)]
        )

    issue_fetch(0, 0)
    issue_fetch(1, 1)

    def pair_step(t, _):
        for b in range(2):
            g = t * 2 + b
            drain_fetch(b)
            extract(g, b)
            gn = g + 2

            @pl.when(gn < _NGRP)
            def _():
                issue_fetch(gn, b)
        return ()

    lax.fori_loop(0, _NGRP // 2, pair_step, (), unroll=False)


@jax.jit
def kernel(input_index, embeds):
    gather = pl.kernel(
        _gather_body,
        mesh=_mesh,
        out_type=jax.ShapeDtypeStruct((_BATCH, _DIM), jnp.float32),
        scratch_types=[
            pltpu.VMEM((_B_PER_W,), jnp.int32),
            pltpu.VMEM((2, _GRP, _DIM, _SLIVER), jnp.float32),
            pltpu.VMEM((2, _GRP, _DIM), jnp.float32),
            pltpu.SemaphoreType.DMA,
            pltpu.SemaphoreType.DMA,
        ],
        compiler_params=pltpu.CompilerParams(
            use_tc_tiling_on_sc=True, needs_layout_passes=False
        ),
    )
    return gather(embeds.T, input_index.astype(jnp.int32))


# speedup vs baseline: 4.0653x; 1.1035x over previous
import jax
import jax.numpy as jnp
from jax import lax
from jax.experimental import pallas as pl
from jax.experimental.pallas import tpu as pltpu
from jax.experimental.pallas import tpu_sc as plsc

_NUM_EMB = 1_000_000
_DIM = 32
_BATCH = 16384
_NUM_CORES = 2
_NUM_SUBCORES = 16
_NUM_WORKERS = _NUM_CORES * _NUM_SUBCORES
_B_PER_W = _BATCH // _NUM_WORKERS
_GRP = 4
_NBUF = 4
_NGRP = _B_PER_W // _GRP
_SLIVER = 128

_mesh = plsc.VectorSubcoreMesh(core_axis_name="c", subcore_axis_name="s")


def _gather_body(table_hbm, idx_hbm, out_hbm, idx_v, fetch_v, stage_v,
                 sem_f0, sem_f1, sem_f2, sem_f3,
                 sem_o0, sem_o1, sem_o2, sem_o3):
    wid = lax.axis_index("s") * _NUM_CORES + lax.axis_index("c")
    base = wid * _B_PER_W
    sems_f = (sem_f0, sem_f1, sem_f2, sem_f3)
    sems_o = (sem_o0, sem_o1, sem_o2, sem_o3)

    pltpu.sync_copy(idx_hbm.at[pl.ds(base, _B_PER_W)], idx_v.at[pl.ds(0, _B_PER_W)])

    rows_lo = lax.iota(jnp.int32, 16)
    rows_hi = rows_lo + 16

    def issue_fetch(g, buf):
        v = idx_v[pl.ds(g * _GRP, 16)]
        j0 = (v >> 7) << 7
        for k in range(_GRP):
            pltpu.async_copy(
                table_hbm.at[
                    pl.ds(0, _DIM),
                    pl.ds(pl.multiple_of(j0[k], 128), _SLIVER),
                ],
                fetch_v.at[buf, k],
                sems_f[buf],
            )

    def drain_fetch(buf):
        for k in range(_GRP):
            pltpu.make_async_copy(
                table_hbm.at[pl.ds(0, _DIM), pl.ds(0, _SLIVER)],
                fetch_v.at[buf, k],
                sems_f[buf],
            ).wait()

    def drain_out(g, buf):
        pltpu.make_async_copy(
            stage_v.at[buf], out_hbm.at[pl.ds(g * _GRP, _GRP)], sems_o[buf]
        ).wait()

    for b in range(_NBUF):
        issue_fetch(b, b)

    def step(t, _):
        for b in range(_NBUF):
            g = t * _NBUF + b
            drain_fetch(b)

            @pl.when(t >= 1)
            def _():
                drain_out(g - _NBUF, b)

            v = idx_v[pl.ds(g * _GRP, 16)]
            lane = v & 127
            for k in range(_GRP):
                sel_k = jnp.full((16,), k, jnp.int32)
                cols = jnp.full((16,), lane[k], jnp.int32)
                lo = plsc.load_gather(fetch_v.at[b], [sel_k, rows_lo, cols])
                hi = plsc.load_gather(fetch_v.at[b], [sel_k, rows_hi, cols])
                stage_v[b, k, pl.ds(0, 16)] = lo
                stage_v[b, k, pl.ds(16, 16)] = hi
            pltpu.async_copy(
                stage_v.at[b],
                out_hbm.at[pl.ds(base + g * _GRP, _GRP)],
                sems_o[b],
            )

            @pl.when(t < (_NGRP // _NBUF) - 1)
            def _():
                issue_fetch(g + _NBUF, b)

        return ()

    lax.fori_loop(0, _NGRP // _NBUF, step, (), unroll=False)

    for b in range(_NBUF):
        drain_out(0, b)


@jax.jit
def kernel(input_index, embeds):
    gather = pl.kernel(
        _gather_body,
        mesh=_mesh,
        out_type=jax.ShapeDtypeStruct((_BATCH, _DIM), jnp.float32),
        scratch_types=[
            pltpu.VMEM((_B_PER_W + 16,), jnp.int32),
            pltpu.VMEM((_NBUF, _GRP, _DIM, _SLIVER), jnp.float32),
            pltpu.VMEM((_NBUF, _GRP, _DIM), jnp.float32),
            pltpu.SemaphoreType.DMA,
            pltpu.SemaphoreType.DMA,
            pltpu.SemaphoreType.DMA,
            pltpu.SemaphoreType.DMA,
            pltpu.SemaphoreType.DMA,
            pltpu.SemaphoreType.DMA,
            pltpu.SemaphoreType.DMA,
            pltpu.SemaphoreType.DMA,
        ],
        compiler_params=pltpu.CompilerParams(
            use_tc_tiling_on_sc=True, needs_layout_passes=False
        ),
    )
    return gather(embeds.T, input_index.astype(jnp.int32))


# speedup vs baseline: 4.3257x; 1.0641x over previous
import jax
import jax.numpy as jnp
from jax import lax
from jax.experimental import pallas as pl
from jax.experimental.pallas import tpu as pltpu
from jax.experimental.pallas import tpu_sc as plsc

_NUM_EMB = 1_000_000
_DIM = 32
_BATCH = 16384
_NUM_CORES = 2
_NUM_SUBCORES = 16
_NUM_WORKERS = _NUM_CORES * _NUM_SUBCORES
_B_PER_W = _BATCH // _NUM_WORKERS
_GRP = 2
_NBUF = 8
_NGRP = _B_PER_W // _GRP
_SLIVER = 128

_mesh = plsc.VectorSubcoreMesh(core_axis_name="c", subcore_axis_name="s")


def _gather_body(table_hbm, idx_hbm, out_hbm, idx_v, fetch_v, stage_v,
                 *sems):
    wid = lax.axis_index("s") * _NUM_CORES + lax.axis_index("c")
    base = wid * _B_PER_W
    sems_f = sems[:_NBUF]
    sems_o = sems[_NBUF:]

    pltpu.sync_copy(idx_hbm.at[pl.ds(base, _B_PER_W)], idx_v.at[pl.ds(0, _B_PER_W)])

    rows_lo = lax.iota(jnp.int32, 16)
    rows_hi = rows_lo + 16

    def issue_fetch(g, buf):
        v = idx_v[pl.ds(g * _GRP, 16)]
        j0 = (v >> 7) << 7
        for k in range(_GRP):
            pltpu.async_copy(
                table_hbm.at[
                    pl.ds(0, _DIM),
                    pl.ds(pl.multiple_of(j0[k], 128), _SLIVER),
                ],
                fetch_v.at[buf, k],
                sems_f[buf],
            )

    def drain_fetch(buf):
        for k in range(_GRP):
            pltpu.make_async_copy(
                table_hbm.at[pl.ds(0, _DIM), pl.ds(0, _SLIVER)],
                fetch_v.at[buf, k],
                sems_f[buf],
            ).wait()

    def drain_out(g, buf):
        pltpu.make_async_copy(
            stage_v.at[buf], out_hbm.at[pl.ds(g * _GRP, _GRP)], sems_o[buf]
        ).wait()

    for b in range(_NBUF):
        issue_fetch(b, b)

    def step(t, _):
        for b in range(_NBUF):
            g = t * _NBUF + b
            drain_fetch(b)

            @pl.when(t >= 1)
            def _():
                drain_out(g - _NBUF, b)

            v = idx_v[pl.ds(g * _GRP, 16)]
            lane = v & 127
            for k in range(_GRP):
                sel_k = jnp.full((16,), k, jnp.int32)
                cols = jnp.full((16,), lane[k], jnp.int32)
                lo = plsc.load_gather(fetch_v.at[b], [sel_k, rows_lo, cols])
                hi = plsc.load_gather(fetch_v.at[b], [sel_k, rows_hi, cols])
                stage_v[b, k, pl.ds(0, 16)] = lo
                stage_v[b, k, pl.ds(16, 16)] = hi
            pltpu.async_copy(
                stage_v.at[b],
                out_hbm.at[pl.ds(base + g * _GRP, _GRP)],
                sems_o[b],
            )

            @pl.when(t < (_NGRP // _NBUF) - 1)
            def _():
                issue_fetch(g + _NBUF, b)

        return ()

    lax.fori_loop(0, _NGRP // _NBUF, step, (), unroll=False)

    for b in range(_NBUF):
        drain_out(0, b)


@jax.jit
def kernel(input_index, embeds):
    gather = pl.kernel(
        _gather_body,
        mesh=_mesh,
        out_type=jax.ShapeDtypeStruct((_BATCH, _DIM), jnp.float32),
        scratch_types=[
            pltpu.VMEM((_B_PER_W + 16,), jnp.int32),
            pltpu.VMEM((_NBUF, _GRP, _DIM, _SLIVER), jnp.float32),
            pltpu.VMEM((_NBUF, _GRP, _DIM), jnp.float32),
        ] + [pltpu.SemaphoreType.DMA] * (2 * _NBUF),
        compiler_params=pltpu.CompilerParams(
            use_tc_tiling_on_sc=True, needs_layout_passes=False
        ),
    )
    return gather(embeds.T, input_index.astype(jnp.int32))
